# Initial kernel scaffold; baseline (speedup 1.0000x reference)
import jax
import jax.numpy as jnp
from jax.experimental import pallas as pl


def kernel(x, edge_index, edge_type, rel_emb, mlp_w, mlp_b, w1, w1_b, w2_src, w2_src_b, w2_dst, w2_dst_b, w4, w4_b, attn):
    raise NotImplementedError("write your pallas kernel here")


# speedup vs baseline: 6.4912x; 6.4912x over previous
import functools

import jax
import jax.numpy as jnp
from jax import lax
from jax.experimental import pallas as pl
from jax.experimental.pallas import tpu as pltpu
from jax.experimental.pallas import tpu_sc as plsc

N = 100000
E = 1600000
D = 128
H = 2
A = 128
HA = H * A
NR = 16

NC = 2
NS = 16
NW = NC * NS

NPAD = 100352
RPT = NPAD // NS

B1 = 80
W1 = (E // NW) // B1
B3 = 400
W3 = (E // NS) // B3

_mesh = plsc.VectorSubcoreMesh(core_axis_name="c", subcore_axis_name="s")


def _iota16():
  return lax.broadcasted_iota(jnp.int32, (16,), 0)


def _bcast(j):
  return jnp.full((16,), 0, jnp.int32) + j


def leaky(x):
  return jnp.maximum(x, 0.2 * x)


_RB = 2000


def _tc_proj_body(x_ref, w2s_ref, w2sb_ref, w2d_ref, w2db_ref, mlp_ref,
                  mlpb_ref, fs_ref, fd_ref, hs_ref):
  xb = x_ref[...]
  fs_ref[...] = jnp.dot(xb, w2s_ref[...],
                        preferred_element_type=jnp.float32) + w2sb_ref[...]
  fd_ref[...] = jnp.dot(xb, w2d_ref[...],
                        preferred_element_type=jnp.float32) + w2db_ref[...]
  t = jnp.dot(xb, mlp_ref[...],
              preferred_element_type=jnp.float32) + mlpb_ref[...]
  hs_ref[...] = leaky(t)


def _tc_proj(x, w2s, w2sb, w2d, w2db, mlp_w, mlpb):
  nblk = N // _RB
  full = lambda s: pl.BlockSpec(s, lambda i: (0, 0))
  row = lambda s: pl.BlockSpec(s, lambda i: (i, 0))
  return pl.pallas_call(
      _tc_proj_body,
      grid=(nblk,),
      in_specs=[row((_RB, D)), full((D, HA)), full((1, HA)), full((D, HA)),
                full((1, HA)), full((D, D)), full((1, D))],
      out_specs=[row((_RB, HA)), row((_RB, HA)), row((_RB, D))],
      out_shape=[jax.ShapeDtypeStruct((N, HA), jnp.float32),
                 jax.ShapeDtypeStruct((N, HA), jnp.float32),
                 jax.ShapeDtypeStruct((N, D), jnp.float32)],
  )(x, w2s, w2sb, w2d, w2db, mlp_w, mlpb)


def _k1_body(fs_hbm, fd_hbm, src_hbm, dst_hbm, attn_hbm,
             p0_hbm, p1_hbm, spart_hbm,
             src_v, dst_v, u_v, attn_b_v, p0_v, p1_v, zb_v, s0_sh, s1_sh,
             sem):
  c = lax.axis_index("c")
  s = lax.axis_index("s")
  wid = c * NS + s

  if True:
    @pl.loop(0, RPT // 16)
    def _z(i):
      zb_v[pl.ds(i * 16, 16)] = jnp.zeros((16,), jnp.float32)
    pltpu.sync_copy(zb_v, s0_sh.at[pl.ds(s * RPT, RPT)])
    pltpu.sync_copy(zb_v, s1_sh.at[pl.ds(s * RPT, RPT)])
    pltpu.sync_copy(attn_hbm, attn_b_v)
    plsc.subcore_barrier()

    tbase = wid * (E // NW)
    iota = _iota16()

    @pl.loop(0, W1)
    def _w(w):
      e0 = tbase + w * B1
      pltpu.sync_copy(src_hbm.at[pl.ds(e0, B1)], src_v)
      pltpu.sync_copy(dst_hbm.at[pl.ds(e0, B1)], dst_v)
      pltpu.async_copy(fs_hbm.at[src_v], u_v, sem).wait()
      pltpu.async_copy(fd_hbm.at[dst_v], u_v, sem, add=True).wait()
      for g in range(B1 // 16):
        rows = iota + g * 16

        def head(lo):
          @pl.loop(0, A, init_carry=jnp.zeros((16,), jnp.float32), unroll=4)
          def acc(j, l):
            u = plsc.load_gather(u_v, [rows, _bcast(lo + j)])
            return l + leaky(u) * attn_b_v[lo + j, :]
          return acc

        l0 = head(0)
        l1 = head(A)
        p0 = jnp.exp(jnp.clip(l0, -60.0, 60.0))
        p1 = jnp.exp(jnp.clip(l1, -60.0, 60.0))
        p0_v[pl.ds(g * 16, 16)] = p0
        p1_v[pl.ds(g * 16, 16)] = p1
      pltpu.sync_copy(p0_v, p0_hbm.at[pl.ds(e0, B1)])
      pltpu.sync_copy(p1_v, p1_hbm.at[pl.ds(e0, B1)])
      pltpu.sync_copy(p0_v, s0_sh.at[dst_v], add=True)
      pltpu.sync_copy(p1_v, s1_sh.at[dst_v], add=True)

    plsc.subcore_barrier()
    rows = pl.ds(s * RPT, RPT)
    pltpu.sync_copy(s0_sh.at[rows], spart_hbm.at[c, 0, rows])
    pltpu.sync_copy(s1_sh.at[rows], spart_hbm.at[c, 1, rows])


@functools.partial(
    pl.kernel,
    out_type=(jax.ShapeDtypeStruct((E,), jnp.float32),
              jax.ShapeDtypeStruct((E,), jnp.float32),
              jax.ShapeDtypeStruct((NC, 2, NPAD), jnp.float32)),
    mesh=_mesh,
    compiler_params=pltpu.CompilerParams(use_tc_tiling_on_sc=False, needs_layout_passes=False),
    scratch_types=[
        pltpu.VMEM((B1,), jnp.int32),
        pltpu.VMEM((B1,), jnp.int32),
        pltpu.VMEM((B1, HA), jnp.float32),
        pltpu.VMEM((HA, 16), jnp.float32),
        pltpu.VMEM((B1,), jnp.float32),
        pltpu.VMEM((B1,), jnp.float32),
        pltpu.VMEM((RPT,), jnp.float32),
        pltpu.VMEM_SHARED((NPAD,), jnp.float32),
        pltpu.VMEM_SHARED((NPAD,), jnp.float32),
        pltpu.SemaphoreType.DMA,
    ],
)
def _k1(*args):
  _k1_body(*args)


def _k2_body(x_hbm, rel_hbm, src_hbm, dst_hbm, et_hbm, p0_hbm, p1_hbm,
             spart_hbm,
             msg_hbm,
             src_v, dst_v, et_v, p0_v, p1_v, sv0_v, sv1_v, x_v, rel_v,
             eb0_v, eb1_v, mb_v, sa_v, sb_v, si_v, i0_sh, i1_sh, sem):
  c = lax.axis_index("c")
  s = lax.axis_index("s")
  wid = c * NS + s

  if True:
    rows = pl.ds(s * RPT, RPT)
    for h, ish in ((0, i0_sh), (1, i1_sh)):
      pltpu.sync_copy(spart_hbm.at[0, h, rows], sa_v)
      pltpu.sync_copy(spart_hbm.at[1, h, rows], sb_v)

      @pl.loop(0, RPT // 16, unroll=4)
      def _inv(i):
        sl = pl.ds(i * 16, 16)
        si_v[sl] = 1.0 / (sa_v[sl] + sb_v[sl] + 1e-9)
      pltpu.sync_copy(si_v, ish.at[rows])
    pltpu.sync_copy(rel_hbm, rel_v)
    plsc.subcore_barrier()

    tbase = wid * (E // NW)
    iota = _iota16()

    @pl.loop(0, W1)
    def _w(w):
      e0 = tbase + w * B1
      pltpu.sync_copy(src_hbm.at[pl.ds(e0, B1)], src_v)
      pltpu.sync_copy(dst_hbm.at[pl.ds(e0, B1)], dst_v)
      pltpu.sync_copy(et_hbm.at[pl.ds(e0, B1)], et_v)
      pltpu.sync_copy(p0_hbm.at[pl.ds(e0, B1)], p0_v)
      pltpu.sync_copy(p1_hbm.at[pl.ds(e0, B1)], p1_v)
      pltpu.async_copy(x_hbm.at[src_v], x_v, sem).wait()
      pltpu.async_copy(i0_sh.at[dst_v], sv0_v, sem).wait()
      pltpu.async_copy(i1_sh.at[dst_v], sv1_v, sem).wait()
      for g in range(B1 // 16):
        gsl = pl.ds(g * 16, 16)
        a0 = p0_v[gsl] * sv0_v[gsl]
        a1 = p1_v[gsl] * sv1_v[gsl]
        rows = iota + g * 16
        relrows = et_v[gsl]

        @pl.loop(0, D, init_carry=(jnp.zeros((16,), jnp.float32),
                                   jnp.zeros((16,), jnp.float32)), unroll=4)
        def pass1(j, carry):
          s0, s1 = carry
          jv = _bcast(j)
          vx = plsc.load_gather(x_v, [rows, jv])
          vr = plsc.load_gather(rel_v, [relrows, jv])
          v = vx * vr
          e0v = jnp.exp(jnp.clip(a0 * v, -80.0, 80.0))
          e1v = jnp.exp(jnp.clip(a1 * v, -80.0, 80.0))
          eb0_v[j, :] = e0v
          eb1_v[j, :] = e1v
          return (s0 + e0v, s1 + e1v)

        s0t, s1t = pass1
        r0 = 1.0 / s0t
        r1 = 1.0 / s1t

        @pl.loop(0, D, unroll=4)
        def pass2(j):
          m = eb0_v[j, :] * r0 + eb1_v[j, :] * r1
          plsc.store_scatter(mb_v, [rows, _bcast(j)], m)

      pltpu.sync_copy(mb_v, msg_hbm.at[pl.ds(e0, B1)])


@functools.partial(
    pl.kernel,
    out_type=jax.ShapeDtypeStruct((E, D), jnp.float32),
    mesh=_mesh,
    compiler_params=pltpu.CompilerParams(use_tc_tiling_on_sc=False, needs_layout_passes=False),
    scratch_types=[
        pltpu.VMEM((B1,), jnp.int32),
        pltpu.VMEM((B1,), jnp.int32),
        pltpu.VMEM((B1,), jnp.int32),
        pltpu.VMEM((B1,), jnp.float32),
        pltpu.VMEM((B1,), jnp.float32),
        pltpu.VMEM((B1,), jnp.float32),
        pltpu.VMEM((B1,), jnp.float32),
        pltpu.VMEM((B1, D), jnp.float32),
        pltpu.VMEM((NR, D), jnp.float32),
        pltpu.VMEM((D, 16), jnp.float32),
        pltpu.VMEM((D, 16), jnp.float32),
        pltpu.VMEM((B1, D), jnp.float32),
        pltpu.VMEM((RPT,), jnp.float32),
        pltpu.VMEM((RPT,), jnp.float32),
        pltpu.VMEM((RPT,), jnp.float32),
        pltpu.VMEM_SHARED((NPAD,), jnp.float32),
        pltpu.VMEM_SHARED((NPAD,), jnp.float32),
        pltpu.SemaphoreType.DMA,
    ],
)
def _k2(*args):
  _k2_body(*args)


def _seg_body(gather_rows, upd_hbm, src_hbm, dst_hbm, out_hbm,
              src_v, dst_v, gidx_v, upd_v, zb_v, acc_sh, sem):
  c = lax.axis_index("c")
  s = lax.axis_index("s")

  if True:
    @pl.loop(0, 112, unroll=4)
    def _z(i):
      zb_v[i, :] = jnp.zeros((16,), jnp.float32)

    iota = _iota16()
    tbase = s * (E // NS)

    for k in range(4):
      chunk = c * 4 + k

      @pl.loop(0, RPT // 112)
      def _zc(i):
        pltpu.sync_copy(zb_v, acc_sh.at[pl.ds(s * RPT + i * 112, 112)])
      plsc.subcore_barrier()

      @pl.loop(0, W3)
      def _w(w):
        e0 = tbase + w * B3
        pltpu.sync_copy(dst_hbm.at[pl.ds(e0, B3)], dst_v)
        if gather_rows:
          pltpu.sync_copy(src_hbm.at[pl.ds(e0, B3)], src_v)

          @pl.loop(0, B3 // 16, unroll=4)
          def _gi(i):
            sl = pl.ds(i * 16, 16)
            gidx_v[sl] = src_v[sl] * 8 + chunk
          pltpu.async_copy(upd_hbm.at[gidx_v], upd_v, sem).wait()
        else:
          pltpu.sync_copy(
              upd_hbm.at[pl.ds(e0, B3), pl.ds(chunk * 16, 16)], upd_v)
        pltpu.sync_copy(upd_v, acc_sh.at[dst_v], add=True)

      plsc.subcore_barrier()
      rows = pl.ds(s * RPT, RPT)
      pltpu.sync_copy(acc_sh.at[rows],
                      out_hbm.at[rows, pl.ds(chunk * 16, 16)])
      plsc.subcore_barrier()


def _make_seg(gather_rows):
  @functools.partial(
      pl.kernel,
      out_type=jax.ShapeDtypeStruct((NPAD, D), jnp.float32),
      mesh=_mesh,
      compiler_params=pltpu.CompilerParams(use_tc_tiling_on_sc=False, needs_layout_passes=False),
      scratch_types=[
          pltpu.VMEM((B3,), jnp.int32),
          pltpu.VMEM((B3,), jnp.int32),
          pltpu.VMEM((B3,), jnp.int32),
          pltpu.VMEM((B3, 16), jnp.float32),
          pltpu.VMEM((112, 16), jnp.float32),
          pltpu.VMEM_SHARED((NPAD, 16), jnp.float32),
          pltpu.SemaphoreType.DMA,
      ],
  )
  def seg(upd_hbm, src_hbm, dst_hbm, out_hbm, *scratch):
    _seg_body(gather_rows, upd_hbm, src_hbm, dst_hbm, out_hbm, *scratch)

  return seg


_k3 = _make_seg(False)
_k4 = _make_seg(True)


def _tc_out_body(x_ref, g_ref, w1_ref, w1b_ref, w4_ref, w4b_ref, o_ref):
  xb = x_ref[...]
  gb = g_ref[...]
  rst = jnp.dot(xb + gb, w1_ref[...],
                preferred_element_type=jnp.float32) + w1b_ref[...]
  inter = jnp.dot(xb * gb, w4_ref[...],
                  preferred_element_type=jnp.float32) + w4b_ref[...]
  o_ref[...] = leaky(rst) + leaky(inter)


def _tc_out(x, g, w1, w1b, w4, w4b):
  nblk = N // _RB
  full = lambda s: pl.BlockSpec(s, lambda i: (0, 0))
  row = lambda s: pl.BlockSpec(s, lambda i: (i, 0))
  return pl.pallas_call(
      _tc_out_body,
      grid=(nblk,),
      in_specs=[row((_RB, D)), row((_RB, D)), full((D, D)), full((1, D)),
                full((D, D)), full((1, D))],
      out_specs=row((_RB, D)),
      out_shape=jax.ShapeDtypeStruct((N, D), jnp.float32),
  )(x, g, w1, w1b, w4, w4b)


def _tc_fin_body(hs_ref, agg_ref, o_ref):
  o_ref[...] = hs_ref[...] + agg_ref[...]


def _tc_fin(hs, agg):
  nblk = N // _RB
  row = lambda s: pl.BlockSpec(s, lambda i: (i, 0))
  return pl.pallas_call(
      _tc_fin_body,
      grid=(nblk,),
      in_specs=[row((_RB, D)), row((_RB, D))],
      out_specs=row((_RB, D)),
      out_shape=jax.ShapeDtypeStruct((N, D), jnp.float32),
  )(hs, agg)


def kernel(x, edge_index, edge_type, rel_emb, mlp_w, mlp_b, w1, w1_b,
           w2_src, w2_src_b, w2_dst, w2_dst_b, w4, w4_b, attn):
  src = edge_index[0].astype(jnp.int32)
  dst = edge_index[1].astype(jnp.int32)
  et = edge_type.astype(jnp.int32)
  attn_bcast = jnp.broadcast_to(attn.reshape((HA, 1)), (HA, 16))

  fs, fd, hs = _tc_proj(x, w2_src, w2_src_b.reshape((1, HA)), w2_dst,
                        w2_dst_b.reshape((1, HA)), mlp_w,
                        mlp_b.reshape((1, D)))
  p0, p1, spart = _k1(fs, fd, src, dst, attn_bcast)
  msg = _k2(x, rel_emb, src, dst, et, p0, p1, spart)
  gagg = _k3(msg, src, dst)
  out = _tc_out(x, gagg[:N], w1, w1_b.reshape((1, D)), w4,
                w4_b.reshape((1, D)))
  agg = _k4(out.reshape((N * 8, 16)), src, dst)
  return _tc_fin(hs, agg[:N])
